# Initial kernel scaffold; baseline (speedup 1.0000x reference)
import jax
import jax.numpy as jnp
from jax.experimental import pallas as pl


def kernel(h, coord_diff, row, col, W1, b1, W2, b2, W3, b3, W4, b4):
    raise NotImplementedError("write your pallas kernel here")


# speedup vs baseline: 1.9738x; 1.9738x over previous
import functools

import jax
import jax.numpy as jnp
from jax import lax
from jax.experimental import pallas as pl
from jax.experimental.pallas import tpu as pltpu
from jax.experimental.pallas import tpu_sc as plsc

F32 = jnp.float32
HIGHEST = lax.Precision.HIGHEST

NC, NS, L = 2, 16, 16
NW = NC * NS
CS = 128


def _dot(a, b):
    return lax.dot_general(a, b, (((1,), (0,)), ((), ())),
                           precision=HIGHEST, preferred_element_type=F32)


def _silu(x):
    return x * jax.nn.sigmoid(x)


def _pre_body(h_ref, w1a_ref, w1b_ref, ha_ref, hb_ref):
    h = h_ref[...]
    ha_ref[...] = _dot(h, w1a_ref[...])
    hb_ref[...] = _dot(h, w1b_ref[...])


def _precompute(h, W1a, W1b, bn):
    N, D = h.shape
    H = W1a.shape[1]
    grid = N // bn
    return pl.pallas_call(
        _pre_body,
        grid=(grid,),
        in_specs=[
            pl.BlockSpec((bn, D), lambda i: (i, 0)),
            pl.BlockSpec((D, H), lambda i: (0, 0)),
            pl.BlockSpec((D, H), lambda i: (0, 0)),
        ],
        out_specs=[
            pl.BlockSpec((bn, H), lambda i: (i, 0)),
            pl.BlockSpec((bn, H), lambda i: (i, 0)),
        ],
        out_shape=[
            jax.ShapeDtypeStruct((N, H), F32),
            jax.ShapeDtypeStruct((N, H), F32),
        ],
    )(h, W1a, W1b)


def _make_gather(N, H, E_pad):
    CPW = E_pad // (NW * CS)
    mesh = plsc.VectorSubcoreMesh(core_axis_name="c", subcore_axis_name="s")

    @functools.partial(
        pl.kernel,
        mesh=mesh,
        out_type=(
            jax.ShapeDtypeStruct((E_pad, H), F32),
            jax.ShapeDtypeStruct((E_pad, H), F32),
        ),
        scratch_types=[
            pltpu.VMEM((CPW, CS), jnp.int32),
            pltpu.VMEM((CPW, CS), jnp.int32),
            pltpu.VMEM((CS, H), F32),
            pltpu.VMEM((CS, H), F32),
            pltpu.SemaphoreType.DMA,
            pltpu.SemaphoreType.DMA,
        ],
        compiler_params=pltpu.CompilerParams(use_tc_tiling_on_sc=False),
    )
    def gather_sc(ha, hb, rows2, cols2, za, zb, idxr, idxc, bufa, bufb, sema, semb):
        cid = lax.axis_index("c")
        sid = lax.axis_index("s")
        wid = sid * NC + cid
        pltpu.sync_copy(rows2.at[pl.ds(wid * CPW, CPW)], idxr)
        pltpu.sync_copy(cols2.at[pl.ds(wid * CPW, CPW)], idxc)

        def body(j, carry):
            off = (wid * CPW + j) * CS
            ca = pltpu.async_copy(ha.at[idxr.at[j]], bufa, sema)
            cb = pltpu.async_copy(hb.at[idxc.at[j]], bufb, semb)
            ca.wait()
            cb.wait()
            pltpu.sync_copy(bufa, za.at[pl.ds(off, CS)])
            pltpu.sync_copy(bufb, zb.at[pl.ds(off, CS)])
            return carry

        lax.fori_loop(0, CPW, body, 0)

    return gather_sc


def _edge_body(E, bE, za_ref, zb_ref, cd_ref, w1c_ref, b1_ref, w2_ref, b2_ref, out_ref):
    i = pl.program_id(0)
    cd = cd_ref[...]
    radial = jnp.sum(cd * cd, axis=1, keepdims=True)
    z = za_ref[...] + zb_ref[...] + radial * w1c_ref[...] + b1_ref[...]
    e1 = _silu(z)
    e2 = _silu(_dot(e1, w2_ref[...]) + b2_ref[...])
    eidx = i * bE + lax.broadcasted_iota(jnp.int32, (bE, 1), 0)
    out_ref[...] = jnp.where(eidx < E, e2, 0.0)


def _edge_mlp(zA, zB, cd_p, w1c, b1, W2, b2, E, bE):
    E_pad, H = zA.shape
    grid = E_pad // bE
    return pl.pallas_call(
        functools.partial(_edge_body, E, bE),
        grid=(grid,),
        in_specs=[
            pl.BlockSpec((bE, H), lambda i: (i, 0)),
            pl.BlockSpec((bE, H), lambda i: (i, 0)),
            pl.BlockSpec((bE, 3), lambda i: (i, 0)),
            pl.BlockSpec((1, H), lambda i: (0, 0)),
            pl.BlockSpec((1, H), lambda i: (0, 0)),
            pl.BlockSpec((H, H), lambda i: (0, 0)),
            pl.BlockSpec((1, H), lambda i: (0, 0)),
        ],
        out_specs=pl.BlockSpec((bE, H), lambda i: (i, 0)),
        out_shape=jax.ShapeDtypeStruct((E_pad, H), F32),
    )(zA, zB, cd_p, w1c, b1, W2, b2)


def _make_scatter(N, H, E_pad):
    CPW = E_pad // (NW * CS)
    RPT = N // NS
    mesh = plsc.VectorSubcoreMesh(core_axis_name="c", subcore_axis_name="s")

    @functools.partial(
        pl.kernel,
        mesh=mesh,
        out_type=jax.ShapeDtypeStruct((NC, N, H), F32),
        scratch_types=[
            pltpu.VMEM((CPW, CS), jnp.int32),
            pltpu.VMEM((CS, H), F32),
            pltpu.VMEM_SHARED((N, H), F32),
        ],
        compiler_params=pltpu.CompilerParams(use_tc_tiling_on_sc=False),
    )
    def scatter_sc(e2, rows2, zeros_nh, out, idxr, ebuf, agg_sh):
        cid = lax.axis_index("c")
        sid = lax.axis_index("s")
        wid = sid * NC + cid
        pltpu.sync_copy(zeros_nh.at[pl.ds(sid * RPT, RPT)],
                        agg_sh.at[pl.ds(sid * RPT, RPT)])
        pltpu.sync_copy(rows2.at[pl.ds(wid * CPW, CPW)], idxr)
        plsc.subcore_barrier()

        def body(j, carry):
            off = (wid * CPW + j) * CS
            pltpu.sync_copy(e2.at[pl.ds(off, CS)], ebuf)
            pltpu.sync_copy(ebuf, agg_sh.at[idxr.at[j]], add=True)
            return carry

        lax.fori_loop(0, CPW, body, 0)
        plsc.subcore_barrier()
        pltpu.sync_copy(agg_sh.at[pl.ds(sid * RPT, RPT)],
                        out.at[cid, pl.ds(sid * RPT, RPT)])

    return scatter_sc


def _node_body(h_ref, a0_ref, a1_ref, w3h_ref, w3a_ref, b3_ref, w4_ref, b4_ref, out_ref):
    agg = a0_ref[...] + a1_ref[...]
    n1 = _silu(_dot(h_ref[...], w3h_ref[...]) + _dot(agg, w3a_ref[...]) + b3_ref[...])
    out_ref[...] = _dot(n1, w4_ref[...]) + b4_ref[...]


def _node_mlp(h, a0, a1, W3h, W3a, b3, W4, b4, bn):
    N, D = h.shape
    H = W3a.shape[0]
    grid = N // bn
    return pl.pallas_call(
        _node_body,
        grid=(grid,),
        in_specs=[
            pl.BlockSpec((bn, D), lambda i: (i, 0)),
            pl.BlockSpec((bn, H), lambda i: (i, 0)),
            pl.BlockSpec((bn, H), lambda i: (i, 0)),
            pl.BlockSpec((D, H), lambda i: (0, 0)),
            pl.BlockSpec((H, H), lambda i: (0, 0)),
            pl.BlockSpec((1, H), lambda i: (0, 0)),
            pl.BlockSpec((H, D), lambda i: (0, 0)),
            pl.BlockSpec((1, D), lambda i: (0, 0)),
        ],
        out_specs=pl.BlockSpec((bn, D), lambda i: (i, 0)),
        out_shape=jax.ShapeDtypeStruct((N, D), F32),
    )(h, a0, a1, W3h, W3a, b3, W4, b4)


def kernel(h, coord_diff, row, col, W1, b1, W2, b2, W3, b3, W4, b4):
    N, D = h.shape
    E = row.shape[0]
    H = W2.shape[0]

    unit = NW * CS * 8
    E_pad = ((E + unit - 1) // unit) * unit
    pad = E_pad - E
    row_p = jnp.concatenate([row, jnp.zeros((pad,), jnp.int32)])
    col_p = jnp.concatenate([col, jnp.zeros((pad,), jnp.int32)])
    cd_p = jnp.concatenate([coord_diff, jnp.zeros((pad, 3), F32)], axis=0)
    rows2 = row_p.reshape(E_pad // CS, CS)
    cols2 = col_p.reshape(E_pad // CS, CS)

    W1a = W1[:D]
    W1b = W1[D:2 * D]
    w1c = W1[2 * D:]
    b1r = b1.reshape(1, H)
    b2r = b2.reshape(1, H)
    W3h = W3[:D]
    W3a = W3[D:]
    b3r = b3.reshape(1, H)
    b4r = b4.reshape(1, D)

    hA, hB = _precompute(h, W1a, W1b, bn=2000)
    zA, zB = _make_gather(N, H, E_pad)(hA, hB, rows2, cols2)
    e2 = _edge_mlp(zA, zB, cd_p, w1c, b1r, W2, b2r, E, bE=2048)
    zeros_nh = jnp.zeros((N, H), F32)
    agg2 = _make_scatter(N, H, E_pad)(e2, rows2, zeros_nh)
    out = _node_mlp(h, agg2[0], agg2[1], W3h, W3a, b3r, W4, b4r, bn=2000)
    return out


# speedup vs baseline: 2.7042x; 1.3700x over previous
import functools

import jax
import jax.numpy as jnp
from jax import lax
from jax.experimental import pallas as pl
from jax.experimental.pallas import tpu as pltpu
from jax.experimental.pallas import tpu_sc as plsc

F32 = jnp.float32
HIGHEST = lax.Precision.HIGHEST

NC, NS = 2, 16
NW = NC * NS
CS = 128
NBUF = 4


def _dot(a, b, dims=(((1,), (0,)), ((), ()))):
    return lax.dot_general(a, b, dims, precision=HIGHEST,
                           preferred_element_type=F32)


def _silu(x):
    return x * jax.nn.sigmoid(x)


def _pre_body(h_ref, w1a_ref, w1b_ref, ha_ref, hb_ref):
    h = h_ref[...]
    ha_ref[...] = _dot(h, w1a_ref[...])
    hb_ref[...] = _dot(h, w1b_ref[...])


def _precompute(h, W1a, W1b, bn):
    N, D = h.shape
    H = W1a.shape[1]
    return pl.pallas_call(
        _pre_body,
        grid=(N // bn,),
        in_specs=[
            pl.BlockSpec((bn, D), lambda i: (i, 0)),
            pl.BlockSpec((D, H), lambda i: (0, 0)),
            pl.BlockSpec((D, H), lambda i: (0, 0)),
        ],
        out_specs=[
            pl.BlockSpec((bn, H), lambda i: (i, 0)),
            pl.BlockSpec((bn, H), lambda i: (i, 0)),
        ],
        out_shape=[
            jax.ShapeDtypeStruct((N, H), F32),
            jax.ShapeDtypeStruct((N, H), F32),
        ],
    )(h, W1a, W1b)


def _make_gather(N, H, E_pad):
    CPW = E_pad // (NW * CS)
    mesh = plsc.VectorSubcoreMesh(core_axis_name="c", subcore_axis_name="s")

    @functools.partial(
        pl.kernel,
        mesh=mesh,
        out_type=(
            jax.ShapeDtypeStruct((E_pad, H), F32),
            jax.ShapeDtypeStruct((E_pad, H), F32),
        ),
        scratch_types=[
            pltpu.VMEM((CPW, CS), jnp.int32),
            pltpu.VMEM((CPW, CS), jnp.int32),
            pltpu.VMEM((NBUF, CS, H), F32),
            pltpu.VMEM((NBUF, CS, H), F32),
            pltpu.SemaphoreType.DMA((NBUF,)),
            pltpu.SemaphoreType.DMA((NBUF,)),
            pltpu.SemaphoreType.DMA((NBUF,)),
            pltpu.SemaphoreType.DMA((NBUF,)),
        ],
        compiler_params=pltpu.CompilerParams(use_tc_tiling_on_sc=False),
    )
    def gather_sc(ha, hb, idxa_h, idxb_h, za, zb,
                  idxa, idxb, bufa, bufb, ga, gb, wa, wb):
        cid = lax.axis_index("c")
        sid = lax.axis_index("s")
        wid = sid * NC + cid
        base = wid * CPW
        pltpu.sync_copy(idxa_h.at[pl.ds(base, CPW)], idxa)
        pltpu.sync_copy(idxb_h.at[pl.ds(base, CPW)], idxb)

        def g_a(j, s):
            return pltpu.make_async_copy(ha.at[idxa.at[j]], bufa.at[s], ga.at[s])

        def g_b(j, s):
            return pltpu.make_async_copy(hb.at[idxb.at[j]], bufb.at[s], gb.at[s])

        def w_a(j, s):
            return pltpu.make_async_copy(
                bufa.at[s], za.at[pl.ds((base + j) * CS, CS)], wa.at[s])

        def w_b(j, s):
            return pltpu.make_async_copy(
                bufb.at[s], zb.at[pl.ds((base + j) * CS, CS)], wb.at[s])

        def body(j, carry):
            s = j % NBUF

            @pl.when(j >= NBUF)
            def _():
                w_a(j - NBUF, s).wait()
                w_b(j - NBUF, s).wait()

            g_a(j, s).start()
            g_b(j, s).start()

            @pl.when(j >= 1)
            def _():
                p = j - 1
                sp = p % NBUF
                g_a(p, sp).wait()
                g_b(p, sp).wait()
                w_a(p, sp).start()
                w_b(p, sp).start()

            return carry

        lax.fori_loop(0, CPW, body, 0)
        p = CPW - 1
        sp = p % NBUF
        g_a(p, sp).wait()
        g_b(p, sp).wait()
        w_a(p, sp).start()
        w_b(p, sp).start()
        for k in range(NBUF):
            q = CPW - NBUF + k
            sq = q % NBUF
            w_a(q, sq).wait()
            w_b(q, sq).wait()

    return gather_sc


def _edge_body(E, Eh, bEh,
               zap_ref, zbp_ref, cdlo_ref, cdhi_ref, w1c3_ref, b1_ref,
               w2d_ref, b2d_ref, out_ref):
    i = pl.program_id(0)
    za = zap_ref[...]
    zb = zbp_ref[...]
    w1c3 = w1c3_ref[...]
    b1 = b1_ref[...]
    cdlo = cdlo_ref[...]
    cdhi = cdhi_ref[...]
    radlo = _dot(cdlo * cdlo, w1c3, (((0,), (0,)), ((), ())))
    radhi = _dot(cdhi * cdhi, w1c3, (((0,), (0,)), ((), ())))
    z1lo = za[:, :64] + zb[:, :64] + radlo + b1
    z1hi = za[:, 64:] + zb[:, 64:] + radhi + b1
    e1 = jnp.concatenate([_silu(z1lo), _silu(z1hi)], axis=1)
    e2 = _silu(_dot(e1, w2d_ref[...]) + b2d_ref[...])
    ridx = lax.broadcasted_iota(jnp.int32, (bEh, 128), 0)
    lane = lax.broadcasted_iota(jnp.int32, (bEh, 128), 1)
    valid = (lane < 64) | (Eh + i * bEh + ridx < E)
    out_ref[...] = jnp.where(valid, e2, 0.0)


def _edge_mlp(zAp, zBp, cdT_p, W1c3, b1r, W2d, b2d, E, Eh, bEh):
    nb = Eh // bEh
    return pl.pallas_call(
        functools.partial(_edge_body, E, Eh, bEh),
        grid=(nb,),
        in_specs=[
            pl.BlockSpec((bEh, 128), lambda i: (i, 0)),
            pl.BlockSpec((bEh, 128), lambda i: (i, 0)),
            pl.BlockSpec((3, bEh), lambda i: (0, i)),
            pl.BlockSpec((3, bEh), lambda i, nb=nb: (0, i + nb)),
            pl.BlockSpec((3, 64), lambda i: (0, 0)),
            pl.BlockSpec((1, 64), lambda i: (0, 0)),
            pl.BlockSpec((128, 128), lambda i: (0, 0)),
            pl.BlockSpec((1, 128), lambda i: (0, 0)),
        ],
        out_specs=pl.BlockSpec((bEh, 128), lambda i: (i, 0)),
        out_shape=jax.ShapeDtypeStruct((Eh, 128), F32),
    )(zAp, zBp, cdT_p, cdT_p, W1c3, b1r, W2d, b2d)


def _make_scatter(N, H, E_pad):
    CPW = E_pad // (NW * CS)
    RPT = N // NS
    mesh = plsc.VectorSubcoreMesh(core_axis_name="c", subcore_axis_name="s")

    @functools.partial(
        pl.kernel,
        mesh=mesh,
        out_type=jax.ShapeDtypeStruct((NC, N, H), F32),
        scratch_types=[
            pltpu.VMEM((CPW, CS), jnp.int32),
            pltpu.VMEM((NBUF, CS, H), F32),
            pltpu.VMEM_SHARED((N, H), F32),
            pltpu.SemaphoreType.DMA((NBUF,)),
        ],
        compiler_params=pltpu.CompilerParams(use_tc_tiling_on_sc=False),
    )
    def scatter_sc(e2, idx_h, zeros_nh, out, idxr, ebuf, acc, lsem):
        cid = lax.axis_index("c")
        sid = lax.axis_index("s")
        wid = sid * NC + cid
        base = wid * CPW
        pltpu.sync_copy(zeros_nh.at[pl.ds(sid * RPT, RPT)],
                        acc.at[pl.ds(sid * RPT, RPT)])
        pltpu.sync_copy(idx_h.at[pl.ds(base, CPW)], idxr)
        plsc.subcore_barrier()

        def load(j, s):
            return pltpu.make_async_copy(
                e2.at[pl.ds((base + j) * CS, CS)], ebuf.at[s], lsem.at[s])

        for k in range(NBUF):
            load(k, k).start()

        def body(j, carry):
            s = j % NBUF
            load(j, s).wait()
            pltpu.sync_copy(ebuf.at[s], acc.at[idxr.at[j]], add=True)

            @pl.when(j + NBUF < CPW)
            def _():
                load(j + NBUF, s).start()

            return carry

        lax.fori_loop(0, CPW, body, 0)
        plsc.subcore_barrier()
        pltpu.sync_copy(acc.at[pl.ds(sid * RPT, RPT)],
                        out.at[cid, pl.ds(sid * RPT, RPT)])

    return scatter_sc


def _node_body(h_ref, a0_ref, a1_ref, w3h_ref, w3a_ref, b3_ref, w4_ref, b4_ref, out_ref):
    agg = a0_ref[...] + a1_ref[...]
    n1 = _silu(_dot(h_ref[...], w3h_ref[...]) + _dot(agg, w3a_ref[...]) + b3_ref[...])
    out_ref[...] = _dot(n1, w4_ref[...]) + b4_ref[...]


def _node_mlp(h, a0, a1, W3h, W3a, b3r, W4, b4r, bn):
    N, D = h.shape
    H = W3a.shape[0]
    return pl.pallas_call(
        _node_body,
        grid=(N // bn,),
        in_specs=[
            pl.BlockSpec((bn, D), lambda i: (i, 0)),
            pl.BlockSpec((bn, H), lambda i: (i, 0)),
            pl.BlockSpec((bn, H), lambda i: (i, 0)),
            pl.BlockSpec((D, H), lambda i: (0, 0)),
            pl.BlockSpec((H, H), lambda i: (0, 0)),
            pl.BlockSpec((1, H), lambda i: (0, 0)),
            pl.BlockSpec((H, D), lambda i: (0, 0)),
            pl.BlockSpec((1, D), lambda i: (0, 0)),
        ],
        out_specs=pl.BlockSpec((bn, D), lambda i: (i, 0)),
        out_shape=jax.ShapeDtypeStruct((N, D), F32),
    )(h, a0, a1, W3h, W3a, b3r, W4, b4r)


def kernel(h, coord_diff, row, col, W1, b1, W2, b2, W3, b3, W4, b4):
    N, D = h.shape
    E = row.shape[0]
    H = W2.shape[0]

    unit = NW * CS * 8
    E_pad = ((E + unit - 1) // unit) * unit
    pad = E_pad - E
    Eh = E_pad // 2

    row_p = jnp.concatenate([row, jnp.zeros((pad,), jnp.int32)])
    col_p = jnp.concatenate([col, jnp.zeros((pad,), jnp.int32)])

    def interleave(x):
        return jnp.stack([x[:Eh], x[Eh:]], axis=1).reshape(-1)

    row_s = interleave(row_p)
    col_s = interleave(col_p)
    idxA2 = row_s.reshape(E_pad // CS, CS)
    idxB2 = col_s.reshape(E_pad // CS, CS)

    W1a = W1[:D]
    W1b = W1[D:2 * D]
    w1c = W1[2 * D:]
    W1c3 = jnp.concatenate([w1c, w1c, w1c], axis=0)
    b1r = b1.reshape(1, H)
    zH = jnp.zeros((H, H), F32)
    W2d = jnp.block([[W2, zH], [zH, W2]])
    b2d = jnp.concatenate([b2, b2]).reshape(1, 2 * H)
    W3h = W3[:D]
    W3a = W3[D:]
    b3r = b3.reshape(1, H)
    b4r = b4.reshape(1, D)

    cdT_p = jnp.pad(coord_diff.T, ((0, 0), (0, pad)))

    hA, hB = _precompute(h, W1a, W1b, bn=2000)
    zA, zB = _make_gather(N, H, E_pad)(hA, hB, idxA2, idxB2)
    zAp = zA.reshape(Eh, 2 * H)
    zBp = zB.reshape(Eh, 2 * H)
    e2p = _edge_mlp(zAp, zBp, cdT_p, W1c3, b1r, W2d, b2d, E, Eh, bEh=2048)
    e2_cat = e2p.reshape(E_pad, H)
    zeros_nh = jnp.zeros((N, H), F32)
    agg2 = _make_scatter(N, H, E_pad)(e2_cat, idxA2, zeros_nh)
    out = _node_mlp(h, agg2[0], agg2[1], W3h, W3a, b3r, W4, b4r, bn=2000)
    return out


# speedup vs baseline: 3.0046x; 1.1111x over previous
import functools

import jax
import jax.numpy as jnp
from jax import lax
from jax.experimental import pallas as pl
from jax.experimental.pallas import tpu as pltpu
from jax.experimental.pallas import tpu_sc as plsc

F32 = jnp.float32
HIGHEST = lax.Precision.HIGHEST

NC, NS = 2, 16
NW = NC * NS
CS = 128
NBUF = 4
NPHASE = 5


def _dot(a, b, dims=(((1,), (0,)), ((), ())), prec=HIGHEST):
    return lax.dot_general(a, b, dims, precision=prec,
                           preferred_element_type=F32)


def _silu(x):
    return x * jax.nn.sigmoid(x)


def _pre_body(h_ref, w1a_ref, w1b_ref, ha_ref, hb_ref):
    h = h_ref[...]
    ha_ref[...] = _dot(h, w1a_ref[...])
    hb_ref[...] = _dot(h, w1b_ref[...])


def _precompute(h, W1a, W1b, bn):
    N, D = h.shape
    H = W1a.shape[1]
    return pl.pallas_call(
        _pre_body,
        grid=(N // bn,),
        in_specs=[
            pl.BlockSpec((bn, D), lambda i: (i, 0)),
            pl.BlockSpec((D, H), lambda i: (0, 0)),
            pl.BlockSpec((D, H), lambda i: (0, 0)),
        ],
        out_specs=[
            pl.BlockSpec((bn, H), lambda i: (i, 0)),
            pl.BlockSpec((bn, H), lambda i: (i, 0)),
        ],
        out_shape=[
            jax.ShapeDtypeStruct((N, H), F32),
            jax.ShapeDtypeStruct((N, H), F32),
        ],
    )(h, W1a, W1b)


def _make_gather(N, H, E_chunk):
    CPW = E_chunk // (NW * CS)
    mesh = plsc.VectorSubcoreMesh(core_axis_name="c", subcore_axis_name="s")

    @functools.partial(
        pl.kernel,
        mesh=mesh,
        out_type=(
            jax.ShapeDtypeStruct((E_chunk, H), F32),
            jax.ShapeDtypeStruct((E_chunk, H), F32),
        ),
        scratch_types=[
            pltpu.VMEM((CPW, CS), jnp.int32),
            pltpu.VMEM((CPW, CS), jnp.int32),
            pltpu.VMEM((NBUF, CS, H), F32),
            pltpu.VMEM((NBUF, CS, H), F32),
            pltpu.SemaphoreType.DMA((NBUF,)),
            pltpu.SemaphoreType.DMA((NBUF,)),
            pltpu.SemaphoreType.DMA((NBUF,)),
            pltpu.SemaphoreType.DMA((NBUF,)),
        ],
        compiler_params=pltpu.CompilerParams(use_tc_tiling_on_sc=False),
    )
    def gather_sc(ha, hb, idxa_h, idxb_h, za, zb,
                  idxa, idxb, bufa, bufb, ga, gb, wa, wb):
        cid = lax.axis_index("c")
        sid = lax.axis_index("s")
        wid = sid * NC + cid
        base = wid * CPW
        pltpu.sync_copy(idxa_h.at[pl.ds(base, CPW)], idxa)
        pltpu.sync_copy(idxb_h.at[pl.ds(base, CPW)], idxb)

        def g_a(j, s):
            return pltpu.make_async_copy(ha.at[idxa.at[j]], bufa.at[s], ga.at[s])

        def g_b(j, s):
            return pltpu.make_async_copy(hb.at[idxb.at[j]], bufb.at[s], gb.at[s])

        def w_a(j, s):
            return pltpu.make_async_copy(
                bufa.at[s], za.at[pl.ds((base + j) * CS, CS)], wa.at[s])

        def w_b(j, s):
            return pltpu.make_async_copy(
                bufb.at[s], zb.at[pl.ds((base + j) * CS, CS)], wb.at[s])

        def body(j, carry):
            s = j % NBUF

            @pl.when(j >= NBUF)
            def _():
                w_a(j - NBUF, s).wait()
                w_b(j - NBUF, s).wait()

            g_a(j, s).start()
            g_b(j, s).start()

            @pl.when(j >= 1)
            def _():
                p = j - 1
                sp = p % NBUF
                g_a(p, sp).wait()
                g_b(p, sp).wait()
                w_a(p, sp).start()
                w_b(p, sp).start()

            return carry

        lax.fori_loop(0, CPW, body, 0)
        p = CPW - 1
        sp = p % NBUF
        g_a(p, sp).wait()
        g_b(p, sp).wait()
        w_a(p, sp).start()
        w_b(p, sp).start()
        for k in range(NBUF):
            q = CPW - NBUF + k
            sq = q % NBUF
            w_a(q, sq).wait()
            w_b(q, sq).wait()

    return gather_sc


def _edge_body(n_pad, n_valid_hi, bEh,
               zap_ref, zbp_ref, cdlo_ref, cdhi_ref, w1c3_ref, b1_ref,
               w2d_ref, b2d_ref, out_ref):
    za = zap_ref[...]
    zb = zbp_ref[...]
    w1c3 = w1c3_ref[...]
    b1 = b1_ref[...]
    cdlo = cdlo_ref[...]
    cdhi = cdhi_ref[...]
    radlo = _dot(cdlo * cdlo, w1c3, (((0,), (0,)), ((), ())))
    radhi = _dot(cdhi * cdhi, w1c3, (((0,), (0,)), ((), ())))
    z1lo = za[:, :64] + zb[:, :64] + radlo + b1
    z1hi = za[:, 64:] + zb[:, 64:] + radhi + b1
    e1 = jnp.concatenate([_silu(z1lo), _silu(z1hi)], axis=1)
    e2 = _silu(_dot(e1, w2d_ref[...]) + b2d_ref[...])
    if n_pad:
        i = pl.program_id(0)
        ridx = i * bEh + lax.broadcasted_iota(jnp.int32, (bEh, 128), 0)
        lane = lax.broadcasted_iota(jnp.int32, (bEh, 128), 1)
        valid = (lane < 64) | (ridx < n_valid_hi)
        out_ref[...] = jnp.where(valid, e2, 0.0)
    else:
        out_ref[...] = e2


def _edge_mlp(zAp, zBp, cd_lo, cd_hi, W1c3, b1r, W2d, b2d, n_pad, bEh):
    Ehp = zAp.shape[0]
    nb = Ehp // bEh
    return pl.pallas_call(
        functools.partial(_edge_body, n_pad, Ehp - n_pad, bEh),
        grid=(nb,),
        in_specs=[
            pl.BlockSpec((bEh, 128), lambda i: (i, 0)),
            pl.BlockSpec((bEh, 128), lambda i: (i, 0)),
            pl.BlockSpec((3, bEh), lambda i: (0, i)),
            pl.BlockSpec((3, bEh), lambda i: (0, i)),
            pl.BlockSpec((3, 64), lambda i: (0, 0)),
            pl.BlockSpec((1, 64), lambda i: (0, 0)),
            pl.BlockSpec((128, 128), lambda i: (0, 0)),
            pl.BlockSpec((1, 128), lambda i: (0, 0)),
        ],
        out_specs=pl.BlockSpec((bEh, 128), lambda i: (i, 0)),
        out_shape=jax.ShapeDtypeStruct((Ehp, 128), F32),
    )(zAp, zBp, cd_lo, cd_hi, W1c3, b1r, W2d, b2d)


def _make_scatter(N, H, E_chunk):
    CPW = E_chunk // (NW * CS)
    RPT = N // NS
    mesh = plsc.VectorSubcoreMesh(core_axis_name="c", subcore_axis_name="s")

    @functools.partial(
        pl.kernel,
        mesh=mesh,
        out_type=jax.ShapeDtypeStruct((NC, N, H), F32),
        scratch_types=[
            pltpu.VMEM((CPW, CS), jnp.int32),
            pltpu.VMEM((NBUF, CS, H), F32),
            pltpu.VMEM_SHARED((N, H), F32),
            pltpu.SemaphoreType.DMA((NBUF,)),
        ],
        compiler_params=pltpu.CompilerParams(use_tc_tiling_on_sc=False),
    )
    def scatter_sc(e2, idx_h, zeros_nh, out, idxr, ebuf, acc, lsem):
        cid = lax.axis_index("c")
        sid = lax.axis_index("s")
        wid = sid * NC + cid
        base = wid * CPW
        pltpu.sync_copy(zeros_nh.at[pl.ds(sid * RPT, RPT)],
                        acc.at[pl.ds(sid * RPT, RPT)])
        pltpu.sync_copy(idx_h.at[pl.ds(base, CPW)], idxr)
        plsc.subcore_barrier()

        def load(j, s):
            return pltpu.make_async_copy(
                e2.at[pl.ds((base + j) * CS, CS)], ebuf.at[s], lsem.at[s])

        for k in range(NBUF):
            load(k, k).start()

        def body(j, carry):
            s = j % NBUF
            load(j, s).wait()
            pltpu.sync_copy(ebuf.at[s], acc.at[idxr.at[j]], add=True)

            @pl.when(j + NBUF < CPW)
            def _():
                load(j + NBUF, s).start()

            return carry

        lax.fori_loop(0, CPW, body, 0)
        plsc.subcore_barrier()
        pltpu.sync_copy(acc.at[pl.ds(sid * RPT, RPT)],
                        out.at[cid, pl.ds(sid * RPT, RPT)])

    return scatter_sc


def _node_body(h_ref, *rest):
    (*agg_refs, w3h_ref, w3a_ref, b3_ref, w4_ref, b4_ref, out_ref) = rest
    agg = agg_refs[0][...]
    for r in agg_refs[1:]:
        agg = agg + r[...]
    n1 = _silu(_dot(h_ref[...], w3h_ref[...]) + _dot(agg, w3a_ref[...]) + b3_ref[...])
    out_ref[...] = _dot(n1, w4_ref[...]) + b4_ref[...]


def _node_mlp(h, aggs, W3h, W3a, b3r, W4, b4r, bn):
    N, D = h.shape
    H = W3a.shape[0]
    return pl.pallas_call(
        _node_body,
        grid=(N // bn,),
        in_specs=[
            pl.BlockSpec((bn, D), lambda i: (i, 0)),
            *[pl.BlockSpec((bn, H), lambda i: (i, 0)) for _ in aggs],
            pl.BlockSpec((D, H), lambda i: (0, 0)),
            pl.BlockSpec((H, H), lambda i: (0, 0)),
            pl.BlockSpec((1, H), lambda i: (0, 0)),
            pl.BlockSpec((H, D), lambda i: (0, 0)),
            pl.BlockSpec((1, D), lambda i: (0, 0)),
        ],
        out_specs=pl.BlockSpec((bn, D), lambda i: (i, 0)),
        out_shape=jax.ShapeDtypeStruct((N, D), F32),
    )(h, *aggs, W3h, W3a, b3r, W4, b4r)


def kernel(h, coord_diff, row, col, W1, b1, W2, b2, W3, b3, W4, b4):
    N, D = h.shape
    E = row.shape[0]
    H = W2.shape[0]

    unit = NPHASE * NW * CS * 8
    E_pad = ((E + unit - 1) // unit) * unit
    pad = E_pad - E
    Eh = E_pad // 2
    E_chunk = E_pad // NPHASE
    Ehp = E_chunk // 2

    row_p = jnp.concatenate([row, jnp.zeros((pad,), jnp.int32)])
    col_p = jnp.concatenate([col, jnp.zeros((pad,), jnp.int32)])

    def interleave(x):
        return jnp.stack([x[:Eh], x[Eh:]], axis=1).reshape(-1)

    row_s = interleave(row_p)
    col_s = interleave(col_p)

    W1a = W1[:D]
    W1b = W1[D:2 * D]
    w1c = W1[2 * D:]
    W1c3 = jnp.concatenate([w1c, w1c, w1c], axis=0)
    b1r = b1.reshape(1, H)
    zH = jnp.zeros((H, H), F32)
    W2d = jnp.block([[W2, zH], [zH, W2]])
    b2d = jnp.concatenate([b2, b2]).reshape(1, 2 * H)
    W3h = W3[:D]
    W3a = W3[D:]
    b3r = b3.reshape(1, H)
    b4r = b4.reshape(1, D)

    cdT_p = jnp.pad(coord_diff.T, ((0, 0), (0, pad)))
    zeros_nh = jnp.zeros((N, H), F32)

    hA, hB = _precompute(h, W1a, W1b, bn=2000)
    gather = _make_gather(N, H, E_chunk)
    scatter = _make_scatter(N, H, E_chunk)

    aggs = []
    for p in range(NPHASE):
        sl = slice(p * E_chunk, (p + 1) * E_chunk)
        idxA2 = row_s[sl].reshape(E_chunk // CS, CS)
        idxB2 = col_s[sl].reshape(E_chunk // CS, CS)
        zA, zB = gather(hA, hB, idxA2, idxB2)
        zAp = zA.reshape(Ehp, 2 * H)
        zBp = zB.reshape(Ehp, 2 * H)
        lo0 = p * Ehp
        hi0 = Eh + p * Ehp
        cd_lo = lax.slice(cdT_p, (0, lo0), (3, lo0 + Ehp))
        cd_hi = lax.slice(cdT_p, (0, hi0), (3, hi0 + Ehp))
        n_pad = min(max(hi0 + Ehp - E, 0), Ehp)
        e2p = _edge_mlp(zAp, zBp, cd_lo, cd_hi, W1c3, b1r, W2d, b2d,
                        n_pad, bEh=2048)
        e2_cat = e2p.reshape(E_chunk, H)
        agg2 = scatter(e2_cat, idxA2, zeros_nh)
        aggs.append(agg2[0])
        aggs.append(agg2[1])

    return _node_mlp(h, aggs, W3h, W3a, b3r, W4, b4r, bn=2000)


# speedup vs baseline: 4.8273x; 1.6066x over previous
import functools

import jax
import jax.numpy as jnp
from jax import lax
from jax.experimental import pallas as pl
from jax.experimental.pallas import tpu as pltpu
from jax.experimental.pallas import tpu_sc as plsc

F32 = jnp.float32
BF16 = jnp.bfloat16
HIGHEST = lax.Precision.HIGHEST

NC, NS = 2, 16
NW = NC * NS
CS = 128
NBUF = 4
NPHASE = 5


def _dot(a, b, dims=(((1,), (0,)), ((), ())), prec=HIGHEST):
    return lax.dot_general(a, b, dims, precision=prec,
                           preferred_element_type=F32)


def _silu(x):
    return x * jax.nn.sigmoid(x)


def _ileave_body(a_ref, b_ref, p_ref, oa_ref, ob_ref):
    p = p_ref[...]
    oa_ref[...] = _dot(a_ref[...].astype(F32), p).astype(jnp.int32)
    ob_ref[...] = _dot(b_ref[...].astype(F32), p).astype(jnp.int32)


def _interleave_stage(rowst, colst, bX):
    X = rowst.shape[0]
    lane = jnp.arange(CS)
    src = lane // 2 + (lane % 2) * (CS // 2)
    P = (jnp.arange(CS)[:, None] == src[None, :]).astype(F32)
    return pl.pallas_call(
        _ileave_body,
        grid=(X // bX,),
        in_specs=[
            pl.BlockSpec((bX, CS), lambda i: (i, 0)),
            pl.BlockSpec((bX, CS), lambda i: (i, 0)),
            pl.BlockSpec((CS, CS), lambda i: (0, 0)),
        ],
        out_specs=[
            pl.BlockSpec((bX, CS), lambda i: (i, 0)),
            pl.BlockSpec((bX, CS), lambda i: (i, 0)),
        ],
        out_shape=[
            jax.ShapeDtypeStruct((X, CS), jnp.int32),
            jax.ShapeDtypeStruct((X, CS), jnp.int32),
        ],
    )(rowst, colst, P)


def _pre_body(h_ref, w1a_ref, w1b_ref, ha_ref, hb_ref):
    h = h_ref[...]
    ha_ref[...] = _dot(h, w1a_ref[...])
    hb_ref[...] = _dot(h, w1b_ref[...])


def _precompute(h, W1a, W1b, bn):
    N, D = h.shape
    H = W1a.shape[1]
    return pl.pallas_call(
        _pre_body,
        grid=(N // bn,),
        in_specs=[
            pl.BlockSpec((bn, D), lambda i: (i, 0)),
            pl.BlockSpec((D, H), lambda i: (0, 0)),
            pl.BlockSpec((D, H), lambda i: (0, 0)),
        ],
        out_specs=[
            pl.BlockSpec((bn, H), lambda i: (i, 0)),
            pl.BlockSpec((bn, H), lambda i: (i, 0)),
        ],
        out_shape=[
            jax.ShapeDtypeStruct((N, H), F32),
            jax.ShapeDtypeStruct((N, H), F32),
        ],
    )(h, W1a, W1b)


def _make_gather(N, H, E_chunk):
    CPW = E_chunk // (NW * CS)
    mesh = plsc.VectorSubcoreMesh(core_axis_name="c", subcore_axis_name="s")

    @functools.partial(
        pl.kernel,
        mesh=mesh,
        out_type=(
            jax.ShapeDtypeStruct((E_chunk, H), F32),
            jax.ShapeDtypeStruct((E_chunk, H), F32),
        ),
        scratch_types=[
            pltpu.VMEM((CPW, CS), jnp.int32),
            pltpu.VMEM((CPW, CS), jnp.int32),
            pltpu.VMEM((NBUF, CS, H), F32),
            pltpu.VMEM((NBUF, CS, H), F32),
            pltpu.SemaphoreType.DMA((NBUF,)),
            pltpu.SemaphoreType.DMA((NBUF,)),
            pltpu.SemaphoreType.DMA((NBUF,)),
            pltpu.SemaphoreType.DMA((NBUF,)),
        ],
        compiler_params=pltpu.CompilerParams(use_tc_tiling_on_sc=False),
    )
    def gather_sc(tok, ha, hb, idxa_h, idxb_h, za, zb,
                  idxa, idxb, bufa, bufb, ga, gb, wa, wb):
        del tok
        cid = lax.axis_index("c")
        sid = lax.axis_index("s")
        wid = sid * NC + cid
        base = wid * CPW
        pltpu.sync_copy(idxa_h.at[pl.ds(base, CPW)], idxa)
        pltpu.sync_copy(idxb_h.at[pl.ds(base, CPW)], idxb)

        def g_a(j, s):
            return pltpu.make_async_copy(ha.at[idxa.at[j]], bufa.at[s], ga.at[s])

        def g_b(j, s):
            return pltpu.make_async_copy(hb.at[idxb.at[j]], bufb.at[s], gb.at[s])

        def w_a(j, s):
            return pltpu.make_async_copy(
                bufa.at[s], za.at[pl.ds((base + j) * CS, CS)], wa.at[s])

        def w_b(j, s):
            return pltpu.make_async_copy(
                bufb.at[s], zb.at[pl.ds((base + j) * CS, CS)], wb.at[s])

        def body(j, carry):
            s = j % NBUF

            @pl.when(j >= NBUF)
            def _():
                w_a(j - NBUF, s).wait()
                w_b(j - NBUF, s).wait()

            g_a(j, s).start()
            g_b(j, s).start()

            @pl.when(j >= 1)
            def _():
                p = j - 1
                sp = p % NBUF
                g_a(p, sp).wait()
                g_b(p, sp).wait()
                w_a(p, sp).start()
                w_b(p, sp).start()

            return carry

        lax.fori_loop(0, CPW, body, 0)
        p = CPW - 1
        sp = p % NBUF
        g_a(p, sp).wait()
        g_b(p, sp).wait()
        w_a(p, sp).start()
        w_b(p, sp).start()
        for k in range(NBUF):
            q = CPW - NBUF + k
            sq = q % NBUF
            w_a(q, sq).wait()
            w_b(q, sq).wait()

    return gather_sc


def _edge_body(n_pad, n_valid_hi, bEh,
               zap_ref, zbp_ref, cdlo_ref, cdhi_ref, w1c3_ref, b1_ref,
               w2d_ref, b2d_ref, out_ref):
    za = zap_ref[...]
    zb = zbp_ref[...]
    w1c3 = w1c3_ref[...]
    b1 = b1_ref[...]
    cdlo = cdlo_ref[...]
    cdhi = cdhi_ref[...]
    radlo = _dot((cdlo * cdlo).astype(BF16), w1c3,
                 (((0,), (0,)), ((), ())), prec=None)
    radhi = _dot((cdhi * cdhi).astype(BF16), w1c3,
                 (((0,), (0,)), ((), ())), prec=None)
    z1lo = za[:, :64] + zb[:, :64] + radlo + b1
    z1hi = za[:, 64:] + zb[:, 64:] + radhi + b1
    e1 = jnp.concatenate([_silu(z1lo), _silu(z1hi)], axis=1)
    e2 = _silu(_dot(e1.astype(BF16), w2d_ref[...], prec=None) + b2d_ref[...])
    if n_pad:
        i = pl.program_id(0)
        ridx = i * bEh + lax.broadcasted_iota(jnp.int32, (bEh, 128), 0)
        lane = lax.broadcasted_iota(jnp.int32, (bEh, 128), 1)
        valid = (lane < 64) | (ridx < n_valid_hi)
        out_ref[...] = jnp.where(valid, e2, 0.0)
    else:
        out_ref[...] = e2


def _edge_mlp(zAp, zBp, cd_lo, cd_hi, W1c3b, b1r, W2db, b2d, n_pad, bEh):
    Ehp = zAp.shape[0]
    nb = Ehp // bEh
    return pl.pallas_call(
        functools.partial(_edge_body, n_pad, Ehp - n_pad, bEh),
        grid=(nb,),
        in_specs=[
            pl.BlockSpec((bEh, 128), lambda i: (i, 0)),
            pl.BlockSpec((bEh, 128), lambda i: (i, 0)),
            pl.BlockSpec((3, bEh), lambda i: (0, i)),
            pl.BlockSpec((3, bEh), lambda i: (0, i)),
            pl.BlockSpec((3, 64), lambda i: (0, 0)),
            pl.BlockSpec((1, 64), lambda i: (0, 0)),
            pl.BlockSpec((128, 128), lambda i: (0, 0)),
            pl.BlockSpec((1, 128), lambda i: (0, 0)),
        ],
        out_specs=pl.BlockSpec((bEh, 128), lambda i: (i, 0)),
        out_shape=jax.ShapeDtypeStruct((Ehp, 128), F32),
    )(zAp, zBp, cd_lo, cd_hi, W1c3b, b1r, W2db, b2d)


def _make_scatter(N, H, E_chunk):
    CPW = E_chunk // (NW * CS)
    RPT = N // NS
    mesh = plsc.VectorSubcoreMesh(core_axis_name="c", subcore_axis_name="s")

    @functools.partial(
        pl.kernel,
        mesh=mesh,
        out_type=(
            jax.ShapeDtypeStruct((N, H), F32),
            jax.ShapeDtypeStruct((N, H), F32),
        ),
        scratch_types=[
            pltpu.VMEM((CPW, CS), jnp.int32),
            pltpu.VMEM((NBUF, CS, H), F32),
            pltpu.VMEM_SHARED((N, H), F32),
            pltpu.SemaphoreType.DMA((NBUF,)),
        ],
        compiler_params=pltpu.CompilerParams(use_tc_tiling_on_sc=False),
    )
    def scatter_sc(tok, e2, idx_h, prev0, prev1, out0, out1,
                   idxr, ebuf, acc, lsem):
        del tok
        cid = lax.axis_index("c")
        sid = lax.axis_index("s")
        wid = sid * NC + cid
        base = wid * CPW
        @pl.when(cid == 0)
        def _():
            pltpu.sync_copy(prev0.at[pl.ds(sid * RPT, RPT)],
                            acc.at[pl.ds(sid * RPT, RPT)])

        @pl.when(cid == 1)
        def _():
            pltpu.sync_copy(prev1.at[pl.ds(sid * RPT, RPT)],
                            acc.at[pl.ds(sid * RPT, RPT)])

        pltpu.sync_copy(idx_h.at[pl.ds(base, CPW)], idxr)
        plsc.subcore_barrier()

        def load(j, s):
            return pltpu.make_async_copy(
                e2.at[pl.ds((base + j) * CS, CS)], ebuf.at[s], lsem.at[s])

        for k in range(NBUF):
            load(k, k).start()

        def body(j, carry):
            s = j % NBUF
            load(j, s).wait()
            pltpu.sync_copy(ebuf.at[s], acc.at[idxr.at[j]], add=True)

            @pl.when(j + NBUF < CPW)
            def _():
                load(j + NBUF, s).start()

            return carry

        lax.fori_loop(0, CPW, body, 0)
        plsc.subcore_barrier()

        @pl.when(cid == 0)
        def _():
            pltpu.sync_copy(acc.at[pl.ds(sid * RPT, RPT)],
                            out0.at[pl.ds(sid * RPT, RPT)])

        @pl.when(cid == 1)
        def _():
            pltpu.sync_copy(acc.at[pl.ds(sid * RPT, RPT)],
                            out1.at[pl.ds(sid * RPT, RPT)])

    return scatter_sc


def _node_body(h_ref, a0_ref, a1_ref, w3h_ref, w3a_ref, b3_ref, w4_ref,
               b4_ref, out_ref):
    agg = a0_ref[...] + a1_ref[...]
    n1 = _silu(_dot(h_ref[...], w3h_ref[...]) + _dot(agg, w3a_ref[...])
               + b3_ref[...])
    out_ref[...] = _dot(n1, w4_ref[...]) + b4_ref[...]


def _node_mlp(h, a0, a1, W3h, W3a, b3r, W4, b4r, bn):
    N, D = h.shape
    H = W3a.shape[0]
    return pl.pallas_call(
        _node_body,
        grid=(N // bn,),
        in_specs=[
            pl.BlockSpec((bn, D), lambda i: (i, 0)),
            pl.BlockSpec((bn, H), lambda i: (i, 0)),
            pl.BlockSpec((bn, H), lambda i: (i, 0)),
            pl.BlockSpec((D, H), lambda i: (0, 0)),
            pl.BlockSpec((H, H), lambda i: (0, 0)),
            pl.BlockSpec((1, H), lambda i: (0, 0)),
            pl.BlockSpec((H, D), lambda i: (0, 0)),
            pl.BlockSpec((1, D), lambda i: (0, 0)),
        ],
        out_specs=pl.BlockSpec((bn, D), lambda i: (i, 0)),
        out_shape=jax.ShapeDtypeStruct((N, D), F32),
    )(h, a0, a1, W3h, W3a, b3r, W4, b4r)


def kernel(h, coord_diff, row, col, W1, b1, W2, b2, W3, b3, W4, b4):
    N, D = h.shape
    E = row.shape[0]
    H = W2.shape[0]

    unit = NPHASE * NW * CS * 8
    E_pad = ((E + unit - 1) // unit) * unit
    pad = E_pad - E
    Eh = E_pad // 2
    E_chunk = E_pad // NPHASE
    Ehp = E_chunk // 2

    row_p = jnp.concatenate([row, jnp.zeros((pad,), jnp.int32)])
    col_p = jnp.concatenate([col, jnp.zeros((pad,), jnp.int32)])

    def stage(x):
        lo = x[:Eh].reshape(E_pad // CS, CS // 2)
        hi = x[Eh:].reshape(E_pad // CS, CS // 2)
        return jnp.concatenate([lo, hi], axis=1)

    rowst, colst = _interleave_stage(stage(row_p), stage(col_p), bX=512)

    W1a = W1[:D]
    W1b = W1[D:2 * D]
    w1c = W1[2 * D:]
    W1c3b = jnp.concatenate([w1c, w1c, w1c], axis=0).astype(BF16)
    b1r = b1.reshape(1, H)
    zH = jnp.zeros((H, H), F32)
    W2db = jnp.block([[W2, zH], [zH, W2]]).astype(BF16)
    b2d = jnp.concatenate([b2, b2]).reshape(1, 2 * H)
    W3h = W3[:D]
    W3a = W3[D:]
    b3r = b3.reshape(1, H)
    b4r = b4.reshape(1, D)

    cdT_p = jnp.pad(coord_diff.T, ((0, 0), (0, pad)))
    zeros_nh = jnp.zeros((N, H), F32)

    hA, hB = _precompute(h, W1a, W1b, bn=2000)
    gather = _make_gather(N, H, E_chunk)
    scatter = _make_scatter(N, H, E_chunk)

    zs = []
    tok = hA
    for p in range(NPHASE):
        sl = slice(p * (E_chunk // CS), (p + 1) * (E_chunk // CS))
        zA, zB = gather(tok, hA, hB, rowst[sl], colst[sl])
        tok = zA
        zs.append((zA, zB))

    e2s = []
    for p in range(NPHASE):
        zA, zB = zs[p]
        zAp = zA.reshape(Ehp, 2 * H)
        zBp = zB.reshape(Ehp, 2 * H)
        lo0 = p * Ehp
        hi0 = Eh + p * Ehp
        cd_lo = lax.slice(cdT_p, (0, lo0), (3, lo0 + Ehp))
        cd_hi = lax.slice(cdT_p, (0, hi0), (3, hi0 + Ehp))
        n_pad = min(max(hi0 + Ehp - E, 0), Ehp)
        e2p = _edge_mlp(zAp, zBp, cd_lo, cd_hi, W1c3b, b1r, W2db, b2d,
                        n_pad, bEh=2048)
        e2s.append(e2p.reshape(E_chunk, H))

    a0, a1 = zeros_nh, zeros_nh
    tok = zs[-1][0]
    for p in range(NPHASE):
        sl = slice(p * (E_chunk // CS), (p + 1) * (E_chunk // CS))
        a0, a1 = scatter(tok, e2s[p], rowst[sl], a0, a1)
        tok = a0

    return _node_mlp(h, a0, a1, W3h, W3a, b3r, W4, b4r, bn=2000)


# speedup vs baseline: 4.8493x; 1.0046x over previous
import functools

import jax
import jax.numpy as jnp
from jax import lax
from jax.experimental import pallas as pl
from jax.experimental.pallas import tpu as pltpu
from jax.experimental.pallas import tpu_sc as plsc

F32 = jnp.float32
BF16 = jnp.bfloat16
HIGHEST = lax.Precision.HIGHEST

NC, NS = 2, 16
NW = NC * NS
CS = 128
NBUF = 4
NPHASE = 5


def _dot(a, b, dims=(((1,), (0,)), ((), ())), prec=HIGHEST):
    return lax.dot_general(a, b, dims, precision=prec,
                           preferred_element_type=F32)


def _silu(x):
    return x * jax.nn.sigmoid(x)


def _ileave_body(a_ref, b_ref, p_ref, oa_ref, ob_ref):
    p = p_ref[...]
    oa_ref[...] = _dot(a_ref[...].astype(F32), p).astype(jnp.int32)
    ob_ref[...] = _dot(b_ref[...].astype(F32), p).astype(jnp.int32)


def _interleave_stage(rowst, colst, bX):
    X = rowst.shape[0]
    lane = jnp.arange(CS)
    src = lane // 2 + (lane % 2) * (CS // 2)
    P = (jnp.arange(CS)[:, None] == src[None, :]).astype(F32)
    return pl.pallas_call(
        _ileave_body,
        grid=(X // bX,),
        in_specs=[
            pl.BlockSpec((bX, CS), lambda i: (i, 0)),
            pl.BlockSpec((bX, CS), lambda i: (i, 0)),
            pl.BlockSpec((CS, CS), lambda i: (0, 0)),
        ],
        out_specs=[
            pl.BlockSpec((bX, CS), lambda i: (i, 0)),
            pl.BlockSpec((bX, CS), lambda i: (i, 0)),
        ],
        out_shape=[
            jax.ShapeDtypeStruct((X, CS), jnp.int32),
            jax.ShapeDtypeStruct((X, CS), jnp.int32),
        ],
    )(rowst, colst, P)


def _pre_body(h_ref, w1a_ref, w1b_ref, ha_ref, hb_ref):
    h = h_ref[...]
    ha_ref[...] = _dot(h, w1a_ref[...])
    hb_ref[...] = _dot(h, w1b_ref[...])


def _precompute(h, W1a, W1b, bn):
    N, D = h.shape
    H = W1a.shape[1]
    return pl.pallas_call(
        _pre_body,
        grid=(N // bn,),
        in_specs=[
            pl.BlockSpec((bn, D), lambda i: (i, 0)),
            pl.BlockSpec((D, H), lambda i: (0, 0)),
            pl.BlockSpec((D, H), lambda i: (0, 0)),
        ],
        out_specs=[
            pl.BlockSpec((bn, H), lambda i: (i, 0)),
            pl.BlockSpec((bn, H), lambda i: (i, 0)),
        ],
        out_shape=[
            jax.ShapeDtypeStruct((N, H), F32),
            jax.ShapeDtypeStruct((N, H), F32),
        ],
    )(h, W1a, W1b)


def _make_gather(N, H, E_chunk):
    CPW = E_chunk // (NW * CS)
    mesh = plsc.VectorSubcoreMesh(core_axis_name="c", subcore_axis_name="s")

    @functools.partial(
        pl.kernel,
        mesh=mesh,
        out_type=(
            jax.ShapeDtypeStruct((E_chunk, H), F32),
            jax.ShapeDtypeStruct((E_chunk, H), F32),
        ),
        scratch_types=[
            pltpu.VMEM((CPW, CS), jnp.int32),
            pltpu.VMEM((CPW, CS), jnp.int32),
            pltpu.VMEM((NBUF, CS, H), F32),
            pltpu.VMEM((NBUF, CS, H), F32),
            pltpu.SemaphoreType.DMA((NBUF,)),
            pltpu.SemaphoreType.DMA((NBUF,)),
            pltpu.SemaphoreType.DMA((NBUF,)),
            pltpu.SemaphoreType.DMA((NBUF,)),
        ],
        compiler_params=pltpu.CompilerParams(use_tc_tiling_on_sc=False),
    )
    def gather_sc(tok, ha, hb, idxa_h, idxb_h, za, zb,
                  idxa, idxb, bufa, bufb, ga, gb, wa, wb):
        del tok
        cid = lax.axis_index("c")
        sid = lax.axis_index("s")
        wid = sid * NC + cid
        base = wid * CPW
        pltpu.sync_copy(idxa_h.at[pl.ds(base, CPW)], idxa)
        pltpu.sync_copy(idxb_h.at[pl.ds(base, CPW)], idxb)

        def g_a(j, s):
            return pltpu.make_async_copy(ha.at[idxa.at[j]], bufa.at[s], ga.at[s])

        def g_b(j, s):
            return pltpu.make_async_copy(hb.at[idxb.at[j]], bufb.at[s], gb.at[s])

        def w_a(j, s):
            return pltpu.make_async_copy(
                bufa.at[s], za.at[pl.ds((base + j) * CS, CS)], wa.at[s])

        def w_b(j, s):
            return pltpu.make_async_copy(
                bufb.at[s], zb.at[pl.ds((base + j) * CS, CS)], wb.at[s])

        def body(j, carry):
            s = j % NBUF

            @pl.when(j >= NBUF)
            def _():
                w_a(j - NBUF, s).wait()
                w_b(j - NBUF, s).wait()

            g_a(j, s).start()
            g_b(j, s).start()

            @pl.when(j >= 1)
            def _():
                p = j - 1
                sp = p % NBUF
                g_a(p, sp).wait()
                g_b(p, sp).wait()
                w_a(p, sp).start()
                w_b(p, sp).start()

            return carry

        lax.fori_loop(0, CPW, body, 0)
        p = CPW - 1
        sp = p % NBUF
        g_a(p, sp).wait()
        g_b(p, sp).wait()
        w_a(p, sp).start()
        w_b(p, sp).start()
        for k in range(NBUF):
            q = CPW - NBUF + k
            sq = q % NBUF
            w_a(q, sq).wait()
            w_b(q, sq).wait()

    return gather_sc


def _edge_body(n_pad, n_valid_hi, bEh,
               zap_ref, zbp_ref, cdlo_ref, cdhi_ref, w1c3_ref, b1_ref,
               w2d_ref, b2d_ref, out_ref):
    za = zap_ref[...]
    zb = zbp_ref[...]
    w1c3 = w1c3_ref[...]
    b1 = b1_ref[...]
    cdlo = cdlo_ref[...]
    cdhi = cdhi_ref[...]
    radlo = _dot((cdlo * cdlo).astype(BF16), w1c3,
                 (((0,), (0,)), ((), ())), prec=None)
    radhi = _dot((cdhi * cdhi).astype(BF16), w1c3,
                 (((0,), (0,)), ((), ())), prec=None)
    z1lo = za[:, :64] + zb[:, :64] + radlo + b1
    z1hi = za[:, 64:] + zb[:, 64:] + radhi + b1
    e1 = jnp.concatenate([_silu(z1lo), _silu(z1hi)], axis=1)
    e2 = _silu(_dot(e1.astype(BF16), w2d_ref[...], prec=None) + b2d_ref[...])
    if n_pad:
        i = pl.program_id(0)
        ridx = i * bEh + lax.broadcasted_iota(jnp.int32, (bEh, 128), 0)
        lane = lax.broadcasted_iota(jnp.int32, (bEh, 128), 1)
        valid = (lane < 64) | (ridx < n_valid_hi)
        out_ref[...] = jnp.where(valid, e2, 0.0)
    else:
        out_ref[...] = e2


def _edge_mlp(zAp, zBp, cd_lo, cd_hi, W1c3b, b1r, W2db, b2d, n_pad, bEh):
    Ehp = zAp.shape[0]
    nb = Ehp // bEh
    return pl.pallas_call(
        functools.partial(_edge_body, n_pad, Ehp - n_pad, bEh),
        grid=(nb,),
        in_specs=[
            pl.BlockSpec((bEh, 128), lambda i: (i, 0)),
            pl.BlockSpec((bEh, 128), lambda i: (i, 0)),
            pl.BlockSpec((3, bEh), lambda i: (0, i)),
            pl.BlockSpec((3, bEh), lambda i: (0, i)),
            pl.BlockSpec((3, 64), lambda i: (0, 0)),
            pl.BlockSpec((1, 64), lambda i: (0, 0)),
            pl.BlockSpec((128, 128), lambda i: (0, 0)),
            pl.BlockSpec((1, 128), lambda i: (0, 0)),
        ],
        out_specs=pl.BlockSpec((bEh, 128), lambda i: (i, 0)),
        out_shape=jax.ShapeDtypeStruct((Ehp, 128), F32),
    )(zAp, zBp, cd_lo, cd_hi, W1c3b, b1r, W2db, b2d)


def _make_scatter(N, H, E_chunk):
    CPW = E_chunk // (NW * CS)
    RPT = N // NS
    mesh = plsc.VectorSubcoreMesh(core_axis_name="c", subcore_axis_name="s")

    @functools.partial(
        pl.kernel,
        mesh=mesh,
        out_type=(
            jax.ShapeDtypeStruct((N, H), F32),
            jax.ShapeDtypeStruct((N, H), F32),
        ),
        scratch_types=[
            pltpu.VMEM((CPW, CS), jnp.int32),
            pltpu.VMEM((NBUF, CS, H), F32),
            pltpu.VMEM_SHARED((N, H), F32),
            pltpu.SemaphoreType.DMA((NBUF,)),
        ],
        compiler_params=pltpu.CompilerParams(use_tc_tiling_on_sc=False),
    )
    def scatter_sc(tok, e2, idx_h, prev0, prev1, out0, out1,
                   idxr, ebuf, acc, lsem):
        del tok
        cid = lax.axis_index("c")
        sid = lax.axis_index("s")
        wid = sid * NC + cid
        base = wid * CPW
        @pl.when(cid == 0)
        def _():
            pltpu.sync_copy(prev0.at[pl.ds(sid * RPT, RPT)],
                            acc.at[pl.ds(sid * RPT, RPT)])

        @pl.when(cid == 1)
        def _():
            pltpu.sync_copy(prev1.at[pl.ds(sid * RPT, RPT)],
                            acc.at[pl.ds(sid * RPT, RPT)])

        pltpu.sync_copy(idx_h.at[pl.ds(base, CPW)], idxr)
        plsc.subcore_barrier()

        def load(j, s):
            return pltpu.make_async_copy(
                e2.at[pl.ds((base + j) * CS, CS)], ebuf.at[s], lsem.at[s])

        for k in range(NBUF):
            load(k, k).start()

        def body(j, carry):
            s = j % NBUF
            load(j, s).wait()
            pltpu.sync_copy(ebuf.at[s], acc.at[idxr.at[j]], add=True)

            @pl.when(j + NBUF < CPW)
            def _():
                load(j + NBUF, s).start()

            return carry

        lax.fori_loop(0, CPW, body, 0)
        plsc.subcore_barrier()

        @pl.when(cid == 0)
        def _():
            pltpu.sync_copy(acc.at[pl.ds(sid * RPT, RPT)],
                            out0.at[pl.ds(sid * RPT, RPT)])

        @pl.when(cid == 1)
        def _():
            pltpu.sync_copy(acc.at[pl.ds(sid * RPT, RPT)],
                            out1.at[pl.ds(sid * RPT, RPT)])

    return scatter_sc


def _node_body(h_ref, a0_ref, a1_ref, w3h_ref, w3a_ref, b3_ref, w4_ref,
               b4_ref, out_ref):
    agg = a0_ref[...] + a1_ref[...]
    n1 = _silu(_dot(h_ref[...], w3h_ref[...]) + _dot(agg, w3a_ref[...])
               + b3_ref[...])
    out_ref[...] = _dot(n1, w4_ref[...]) + b4_ref[...]


def _node_mlp(h, a0, a1, W3h, W3a, b3r, W4, b4r, bn):
    N, D = h.shape
    H = W3a.shape[0]
    return pl.pallas_call(
        _node_body,
        grid=(N // bn,),
        in_specs=[
            pl.BlockSpec((bn, D), lambda i: (i, 0)),
            pl.BlockSpec((bn, H), lambda i: (i, 0)),
            pl.BlockSpec((bn, H), lambda i: (i, 0)),
            pl.BlockSpec((D, H), lambda i: (0, 0)),
            pl.BlockSpec((H, H), lambda i: (0, 0)),
            pl.BlockSpec((1, H), lambda i: (0, 0)),
            pl.BlockSpec((H, D), lambda i: (0, 0)),
            pl.BlockSpec((1, D), lambda i: (0, 0)),
        ],
        out_specs=pl.BlockSpec((bn, D), lambda i: (i, 0)),
        out_shape=jax.ShapeDtypeStruct((N, D), F32),
    )(h, a0, a1, W3h, W3a, b3r, W4, b4r)


def kernel(h, coord_diff, row, col, W1, b1, W2, b2, W3, b3, W4, b4):
    N, D = h.shape
    E = row.shape[0]
    H = W2.shape[0]

    unit = NPHASE * NW * CS * 8
    E_pad = ((E + unit - 1) // unit) * unit
    pad = E_pad - E
    Eh = E_pad // 2
    E_chunk = E_pad // NPHASE
    Ehp = E_chunk // 2

    row_p = jnp.concatenate([row, jnp.zeros((pad,), jnp.int32)])
    col_p = jnp.concatenate([col, jnp.zeros((pad,), jnp.int32)])

    def stage(x):
        lo = x[:Eh].reshape(E_pad // CS, CS // 2)
        hi = x[Eh:].reshape(E_pad // CS, CS // 2)
        return jnp.concatenate([lo, hi], axis=1)

    rowst, colst = _interleave_stage(stage(row_p), stage(col_p), bX=512)

    W1a = W1[:D]
    W1b = W1[D:2 * D]
    w1c = W1[2 * D:]
    W1c3b = jnp.concatenate([w1c, w1c, w1c], axis=0).astype(BF16)
    b1r = b1.reshape(1, H)
    zH = jnp.zeros((H, H), F32)
    W2db = jnp.block([[W2, zH], [zH, W2]]).astype(BF16)
    b2d = jnp.concatenate([b2, b2]).reshape(1, 2 * H)
    W3h = W3[:D]
    W3a = W3[D:]
    b3r = b3.reshape(1, H)
    b4r = b4.reshape(1, D)

    cdT_p = jnp.pad(coord_diff.T, ((0, 0), (0, pad)))
    zeros_nh = jnp.zeros((N, H), F32)

    hA, hB = _precompute(h, W1a, W1b, bn=2000)
    gather = _make_gather(N, H, E_chunk)
    scatter = _make_scatter(N, H, E_chunk)

    zs = []
    e2s = []
    tok = hA
    for p in range(NPHASE):
        sl = slice(p * (E_chunk // CS), (p + 1) * (E_chunk // CS))
        zA, zB = gather(tok, hA, hB, rowst[sl], colst[sl])
        tok = zA
        zs.append((zA, zB))
        zAp = zA.reshape(Ehp, 2 * H)
        zBp = zB.reshape(Ehp, 2 * H)
        lo0 = p * Ehp
        hi0 = Eh + p * Ehp
        cd_lo = lax.slice(cdT_p, (0, lo0), (3, lo0 + Ehp))
        cd_hi = lax.slice(cdT_p, (0, hi0), (3, hi0 + Ehp))
        n_pad = min(max(hi0 + Ehp - E, 0), Ehp)
        e2p = _edge_mlp(zAp, zBp, cd_lo, cd_hi, W1c3b, b1r, W2db, b2d,
                        n_pad, bEh=2048)
        e2s.append(e2p.reshape(E_chunk, H))

    a0, a1 = zeros_nh, zeros_nh
    tok = zs[-1][0]
    for p in range(NPHASE):
        sl = slice(p * (E_chunk // CS), (p + 1) * (E_chunk // CS))
        a0, a1 = scatter(tok, e2s[p], rowst[sl], a0, a1)
        tok = a0

    return _node_mlp(h, a0, a1, W3h, W3a, b3r, W4, b4r, bn=2000)
